# baseline (speedup 1.0000x reference)
import jax
import jax.numpy as jnp
from jax.experimental import pallas as pl
from jax.experimental.pallas import tpu as pltpu

_EPS = 1e-5
_NSTREAM = 5
_BQ = 40
_BOUT = 1000


def _bn_affine(stats, gamma, beta, n_rows):
    mu = stats[0:1, :] / n_rows
    var = stats[1:2, :] / n_rows - mu * mu
    s = gamma * jax.lax.rsqrt(var + _EPS)
    t = beta - mu * s
    return s, t


def _make_fused_kernel(n, p, d_h, d_out):
    rows_per_step = _NSTREAM * _BQ

    def fused(*refs):
        sup_refs = refs[:_NSTREAM]
        (x_ref, w1_ref, w2_ref, b1_ref, g1_ref, be1_ref,
         b2_ref, g2_ref, be2_ref, out_ref,
         a_s, b_s, st1_s, st2_s) = refs[_NSTREAM:]
        i = pl.program_id(0)

        @pl.when(i == 0)
        def _():
            a_s[...] = jnp.dot(
                x_ref[...], w1_ref[...],
                preferred_element_type=jnp.float32,
            )

        @pl.when(i < p)
        def _():
            st = jnp.zeros((2, d_h), jnp.float32)
            for j, sref in enumerate(sup_refs):
                a = jnp.dot(sref[...], a_s[...],
                            preferred_element_type=jnp.float32)
                z = jnp.maximum(a + b1_ref[...], 0.0)
                b_s[pl.ds(i * rows_per_step + j * _BQ, _BQ), :] = z
                st = st + jnp.concatenate(
                    [jnp.sum(z, axis=0, keepdims=True),
                     jnp.sum(z * z, axis=0, keepdims=True)], axis=0)

            @pl.when(i == 0)
            def _():
                st1_s[...] = st

            @pl.when(i != 0)
            def _():
                st1_s[...] += st

        @pl.when(jnp.logical_and(i >= p, i < 2 * p))
        def _():
            @pl.when(i == p)
            def _():
                s1, t1 = _bn_affine(st1_s[...], g1_ref[...], be1_ref[...], n)
                h = b_s[...] * s1 + t1
                a_s[:, 0:d_out] = jnp.dot(
                    h, w2_ref[...], preferred_element_type=jnp.float32)

            st = jnp.zeros((2, d_out), jnp.float32)
            for j, sref in enumerate(sup_refs):
                a = jnp.dot(sref[...], a_s[:, 0:d_out],
                            preferred_element_type=jnp.float32)
                y = jnp.maximum(a + b2_ref[...], 0.0)
                b_s[pl.ds((i - p) * rows_per_step + j * _BQ, _BQ), 0:d_out] = y
                st = st + jnp.concatenate(
                    [jnp.sum(y, axis=0, keepdims=True),
                     jnp.sum(y * y, axis=0, keepdims=True)], axis=0)

            @pl.when(i == p)
            def _():
                st2_s[...] = st

            @pl.when(i != p)
            def _():
                st2_s[...] += st

        @pl.when(i >= 2 * p)
        def _():
            s2, t2 = _bn_affine(st2_s[...], g2_ref[...], be2_ref[...], n)
            yb = b_s[pl.ds((i - 2 * p) * _BOUT, _BOUT), 0:d_out]
            out_ref[...] = yb * s2 + t2

    return fused


def kernel(x, support, W1, b1, gamma1, beta1, W2, b2, gamma2, beta2):
    n, d_in = x.shape
    d_h = W1.shape[1]
    d_out = W2.shape[1]
    rows_per_step = _NSTREAM * _BQ
    assert n % rows_per_step == 0 and n % _BOUT == 0
    p = n // rows_per_step
    q = n // _BOUT

    def make_sup_idx(j):
        def sup_idx(i):
            step = jnp.where(i < p, i, jnp.where(i < 2 * p, i - p, p - 1))
            return (step * _NSTREAM + j, 0)
        return sup_idx

    def out_idx(i):
        return (jnp.where(i < 2 * p, 0, i - 2 * p), 0)

    const = lambda i: (0, 0)

    out = pl.pallas_call(
        _make_fused_kernel(n, p, d_h, d_out),
        grid=(2 * p + q,),
        in_specs=(
            [pl.BlockSpec((_BQ, n), make_sup_idx(j)) for j in range(_NSTREAM)]
            + [
                pl.BlockSpec((n, d_in), const),
                pl.BlockSpec((d_in, d_h), const),
                pl.BlockSpec((d_h, d_out), const),
                pl.BlockSpec((1, d_h), const),
                pl.BlockSpec((1, d_h), const),
                pl.BlockSpec((1, d_h), const),
                pl.BlockSpec((1, d_out), const),
                pl.BlockSpec((1, d_out), const),
                pl.BlockSpec((1, d_out), const),
            ]
        ),
        out_specs=pl.BlockSpec((_BOUT, d_out), out_idx),
        out_shape=jax.ShapeDtypeStruct((n, d_out), jnp.float32),
        scratch_shapes=[
            pltpu.VMEM((n, d_h), jnp.float32),
            pltpu.VMEM((n, d_h), jnp.float32),
            pltpu.VMEM((2, d_h), jnp.float32),
            pltpu.VMEM((2, d_out), jnp.float32),
        ],
    )(*([support] * _NSTREAM), x, W1, W2,
      b1.reshape(1, d_h), gamma1.reshape(1, d_h), beta1.reshape(1, d_h),
      b2.reshape(1, d_out), gamma2.reshape(1, d_out), beta2.reshape(1, d_out))

    return (out, support)


# speedup vs baseline: 1.2294x; 1.2294x over previous
import jax
import jax.numpy as jnp
from jax.experimental import pallas as pl
from jax.experimental.pallas import tpu as pltpu

_EPS = 1e-5
_NSTREAM = 2
_BQ = 200
_BOUT = 1000


def _bn_affine(stats, gamma, beta, n_rows):
    mu = stats[0:1, :] / n_rows
    var = stats[1:2, :] / n_rows - mu * mu
    s = gamma * jax.lax.rsqrt(var + _EPS)
    t = beta - mu * s
    return s, t


def _make_fused_kernel(n, p, d_h, d_out):
    rows_per_step = _NSTREAM * _BQ

    def fused(*refs):
        sup_refs = refs[:_NSTREAM]
        (x_ref, w1_ref, w2_ref, b1_ref, g1_ref, be1_ref,
         b2_ref, g2_ref, be2_ref, out_ref,
         a_s, b_s, st1_s, st2_s) = refs[_NSTREAM:]
        i = pl.program_id(0)

        @pl.when(i == 0)
        def _():
            a_s[...] = jnp.dot(
                x_ref[...], w1_ref[...],
                preferred_element_type=jnp.float32,
            )

        @pl.when(i < p)
        def _():
            st = jnp.zeros((2, d_h), jnp.float32)
            for j, sref in enumerate(sup_refs):
                a = jnp.dot(sref[...], a_s[...],
                            preferred_element_type=jnp.float32)
                z = jnp.maximum(a + b1_ref[...], 0.0)
                b_s[pl.ds(i * rows_per_step + j * _BQ, _BQ), :] = z
                st = st + jnp.concatenate(
                    [jnp.sum(z, axis=0, keepdims=True),
                     jnp.sum(z * z, axis=0, keepdims=True)], axis=0)

            @pl.when(i == 0)
            def _():
                st1_s[...] = st

            @pl.when(i != 0)
            def _():
                st1_s[...] += st

        @pl.when(jnp.logical_and(i >= p, i < 2 * p))
        def _():
            @pl.when(i == p)
            def _():
                s1, t1 = _bn_affine(st1_s[...], g1_ref[...], be1_ref[...], n)
                h = b_s[...] * s1 + t1
                a_s[:, 0:d_out] = jnp.dot(
                    h, w2_ref[...], preferred_element_type=jnp.float32)

            st = jnp.zeros((2, d_out), jnp.float32)
            for j, sref in enumerate(sup_refs):
                a = jnp.dot(sref[...], a_s[:, 0:d_out],
                            preferred_element_type=jnp.float32)
                y = jnp.maximum(a + b2_ref[...], 0.0)
                b_s[pl.ds((i - p) * rows_per_step + j * _BQ, _BQ), 0:d_out] = y
                st = st + jnp.concatenate(
                    [jnp.sum(y, axis=0, keepdims=True),
                     jnp.sum(y * y, axis=0, keepdims=True)], axis=0)

            @pl.when(i == p)
            def _():
                st2_s[...] = st

            @pl.when(i != p)
            def _():
                st2_s[...] += st

        @pl.when(i >= 2 * p)
        def _():
            s2, t2 = _bn_affine(st2_s[...], g2_ref[...], be2_ref[...], n)
            yb = b_s[pl.ds((i - 2 * p) * _BOUT, _BOUT), 0:d_out]
            out_ref[...] = yb * s2 + t2

    return fused


def kernel(x, support, W1, b1, gamma1, beta1, W2, b2, gamma2, beta2):
    n, d_in = x.shape
    d_h = W1.shape[1]
    d_out = W2.shape[1]
    rows_per_step = _NSTREAM * _BQ
    assert n % rows_per_step == 0 and n % _BOUT == 0
    p = n // rows_per_step
    q = n // _BOUT

    def make_sup_idx(j):
        def sup_idx(i):
            step = jnp.where(i < p, i, jnp.where(i < 2 * p, i - p, p - 1))
            return (step * _NSTREAM + j, 0)
        return sup_idx

    def out_idx(i):
        return (jnp.where(i < 2 * p, 0, i - 2 * p), 0)

    const = lambda i: (0, 0)

    out = pl.pallas_call(
        _make_fused_kernel(n, p, d_h, d_out),
        grid=(2 * p + q,),
        in_specs=(
            [pl.BlockSpec((_BQ, n), make_sup_idx(j)) for j in range(_NSTREAM)]
            + [
                pl.BlockSpec((n, d_in), const),
                pl.BlockSpec((d_in, d_h), const),
                pl.BlockSpec((d_h, d_out), const),
                pl.BlockSpec((1, d_h), const),
                pl.BlockSpec((1, d_h), const),
                pl.BlockSpec((1, d_h), const),
                pl.BlockSpec((1, d_out), const),
                pl.BlockSpec((1, d_out), const),
                pl.BlockSpec((1, d_out), const),
            ]
        ),
        out_specs=pl.BlockSpec((_BOUT, d_out), out_idx),
        out_shape=jax.ShapeDtypeStruct((n, d_out), jnp.float32),
        scratch_shapes=[
            pltpu.VMEM((n, d_h), jnp.float32),
            pltpu.VMEM((n, d_h), jnp.float32),
            pltpu.VMEM((2, d_h), jnp.float32),
            pltpu.VMEM((2, d_out), jnp.float32),
        ],
    )(*([support] * _NSTREAM), x, W1, W2,
      b1.reshape(1, d_h), gamma1.reshape(1, d_h), beta1.reshape(1, d_h),
      b2.reshape(1, d_out), gamma2.reshape(1, d_out), beta2.reshape(1, d_out))

    return (out, support)
